# baseline (speedup 1.0000x reference)
import functools

import jax
import jax.numpy as jnp
from jax import lax
from jax.experimental import pallas as pl
from jax.experimental.pallas import tpu as pltpu
from jax.experimental.pallas import tpu_sc as plsc

DIM = 32
NROW = 16384
NCOL = 26
NUM_INDICES = NROW * NCOL
NUM_CORES = 2
NUM_SUBCORES = 16
NW = NUM_CORES * NUM_SUBCORES
B_PER_W = NUM_INDICES // NW
BLK = 128
GBLK = 4
SLOT = GBLK * BLK
NCH = B_PER_W // SLOT
BLOCKS_PER_W = B_PER_W // BLK
NBUF = 2

_mesh = plsc.VectorSubcoreMesh(core_axis_name="c", subcore_axis_name="s")


@functools.partial(
    pl.kernel,
    mesh=_mesh,
    out_type=jax.ShapeDtypeStruct((NCOL, DIM // 8, NROW // 128, 8, 128),
                                  jnp.float32),
    scratch_types=[
        pltpu.VMEM((NBUF, SLOT), jnp.int32),
        pltpu.VMEM((NBUF, SLOT, DIM), jnp.float32),
        pltpu.VMEM((NBUF, DIM // 8, GBLK, 8, BLK), jnp.float32),
        pltpu.SemaphoreType.DMA,
        pltpu.SemaphoreType.DMA,
        pltpu.SemaphoreType.DMA,
        pltpu.SemaphoreType.DMA,
    ],
    compiler_params=pltpu.CompilerParams(
        use_tc_tiling_on_sc=False, needs_layout_passes=False),
)
def _gather_kernel(idx_hbm, table_hbm, out_hbm, idx_v, gath, cbuf,
                   g0, g1, w0, w1):
    gsems = (g0, g1)
    wsems = (w0, w1)

    wid = lax.axis_index("s") * NUM_CORES + lax.axis_index("c")
    base = wid * B_PER_W
    iota = lax.iota(jnp.int32, 16)

    def stage(i, b):
        pltpu.sync_copy(idx_hbm.at[pl.ds(base + i * SLOT, SLOT)], idx_v.at[b])
        pltpu.async_copy(table_hbm.at[idx_v.at[b]], gath.at[b], gsems[b])

    def wb_descr(i, b):
        block0 = wid * BLOCKS_PER_W + i * GBLK
        j = block0 // 128
        b1 = block0 % 128
        return pltpu.make_async_copy(
            cbuf.at[b], out_hbm.at[j, :, pl.ds(b1, GBLK)], wsems[b])

    def visit(i, b):
        pltpu.make_async_copy(
            table_hbm.at[idx_v.at[b]], gath.at[b], gsems[b]).wait()
        gb = gath.at[b]
        cb = cbuf.at[b]

        @pl.when(i >= NBUF)
        def _():
            wb_descr(0, b).wait()

        for kk in range(GBLK):
            @plsc.parallel_loop(0, DIM * (BLK // 16), unroll=16)
            def _transpose(t):
                g16 = lax.shift_left(lax.shift_right_logical(t, 5), 4)
                c0 = lax.bitwise_and(t, 31)
                rows = kk * BLK + g16 + iota
                cvec = lax.bitwise_and(c0 + iota, 31)
                vals = plsc.load_gather(gb, [rows, cvec])
                plsc.store_scatter(
                    cb,
                    [lax.shift_right_logical(cvec, 3),
                     jnp.broadcast_to(kk, (16,)),
                     lax.bitwise_and(cvec, 7),
                     g16 + iota],
                    vals)

        block0 = wid * BLOCKS_PER_W + i * GBLK
        j = block0 // 128
        b1 = block0 % 128
        pltpu.async_copy(cb, out_hbm.at[j, :, pl.ds(b1, GBLK)], wsems[b])

        @pl.when(i + NBUF < NCH)
        def _():
            stage(i + NBUF, b)

    for b in range(NBUF):
        stage(b, b)

    def body(jj, carry):
        for b in range(NBUF):
            visit(jj * NBUF + b, b)
        return carry

    lax.fori_loop(0, NCH // NBUF, body, 0)

    for b in range(NBUF):
        wb_descr(0, b).wait()


def kernel(x, table):
    idx = jnp.transpose(x).reshape(-1)
    o5 = _gather_kernel(idx, table)
    return jnp.transpose(o5, (2, 4, 0, 1, 3)).reshape(NROW, NCOL, DIM)


# speedup vs baseline: 1.0042x; 1.0042x over previous
import functools

import jax
import jax.numpy as jnp
from jax import lax
from jax.experimental import pallas as pl
from jax.experimental.pallas import tpu as pltpu
from jax.experimental.pallas import tpu_sc as plsc

DIM = 32
NROW = 16384
NCOL = 26
NUM_INDICES = NROW * NCOL
NUM_CORES = 2
NUM_SUBCORES = 16
NW = NUM_CORES * NUM_SUBCORES
B_PER_W = NUM_INDICES // NW
BLK = 128
GBLK = 4
SLOT = GBLK * BLK
NCH = B_PER_W // SLOT
BLOCKS_PER_W = B_PER_W // BLK
NBUF = 3

_mesh = plsc.VectorSubcoreMesh(core_axis_name="c", subcore_axis_name="s")


@functools.partial(
    pl.kernel,
    mesh=_mesh,
    out_type=jax.ShapeDtypeStruct((NCOL, DIM // 8, NROW // 128, 8, 128),
                                  jnp.float32),
    scratch_types=[
        pltpu.VMEM((NBUF, SLOT), jnp.int32),
        pltpu.VMEM((NBUF, SLOT, DIM), jnp.float32),
        pltpu.VMEM((NBUF, DIM // 8, GBLK, 8, BLK), jnp.float32),
        pltpu.SemaphoreType.DMA,
        pltpu.SemaphoreType.DMA,
        pltpu.SemaphoreType.DMA,
        pltpu.SemaphoreType.DMA,
        pltpu.SemaphoreType.DMA,
        pltpu.SemaphoreType.DMA,
    ],
    compiler_params=pltpu.CompilerParams(
        use_tc_tiling_on_sc=False, needs_layout_passes=False),
)
def _gather_kernel(idx_hbm, table_hbm, out_hbm, idx_v, gath, cbuf,
                   g0, g1, g2, w0, w1, w2):
    gsems = (g0, g1, g2)
    wsems = (w0, w1, w2)

    wid = lax.axis_index("s") * NUM_CORES + lax.axis_index("c")
    base = wid * B_PER_W
    iota = lax.iota(jnp.int32, 16)

    def stage(i, b):
        pltpu.sync_copy(idx_hbm.at[pl.ds(base + i * SLOT, SLOT)], idx_v.at[b])
        pltpu.async_copy(table_hbm.at[idx_v.at[b]], gath.at[b], gsems[b])

    def wb_descr(i, b):
        block0 = wid * BLOCKS_PER_W + i * GBLK
        j = block0 // 128
        b1 = block0 % 128
        return pltpu.make_async_copy(
            cbuf.at[b], out_hbm.at[j, :, pl.ds(b1, GBLK)], wsems[b])

    def visit(i, b):
        pltpu.make_async_copy(
            table_hbm.at[idx_v.at[b]], gath.at[b], gsems[b]).wait()
        gb = gath.at[b]
        cb = cbuf.at[b]

        @pl.when(i >= NBUF)
        def _():
            wb_descr(0, b).wait()

        for kk in range(GBLK):
            @plsc.parallel_loop(0, DIM * (BLK // 16), unroll=8)
            def _transpose(t):
                g16 = lax.shift_left(lax.shift_right_logical(t, 5), 4)
                c0 = lax.bitwise_and(t, 31)
                rows = kk * BLK + g16 + iota
                cvec = lax.bitwise_and(c0 + iota, 31)
                vals = plsc.load_gather(gb, [rows, cvec])
                plsc.store_scatter(
                    cb,
                    [lax.shift_right_logical(cvec, 3),
                     jnp.broadcast_to(kk, (16,)),
                     lax.bitwise_and(cvec, 7),
                     g16 + iota],
                    vals)

        block0 = wid * BLOCKS_PER_W + i * GBLK
        j = block0 // 128
        b1 = block0 % 128
        pltpu.async_copy(cb, out_hbm.at[j, :, pl.ds(b1, GBLK)], wsems[b])

        @pl.when(i + NBUF < NCH)
        def _():
            stage(i + NBUF, b)

    for b in range(NBUF):
        stage(b, b)

    def body(jj, carry):
        for b in range(NBUF):
            visit(jj * NBUF + b, b)
        return carry

    lax.fori_loop(0, NCH // NBUF, body, 0)
    for t in range(NCH - NCH // NBUF * NBUF):
        visit(NCH // NBUF * NBUF + t, t)

    for b in range(NBUF):
        wb_descr(0, b).wait()


def kernel(x, table):
    idx = jnp.transpose(x).reshape(-1)
    o5 = _gather_kernel(idx, table)
    return jnp.transpose(o5, (2, 4, 0, 1, 3)).reshape(NROW, NCOL, DIM)
